# baseline (speedup 1.0000x reference)
import jax
import jax.numpy as jnp
from jax.experimental import pallas as pl

N_NODES = 10000
N_EDGES = 160000


def _ln_kernel(x_ref, g_ref, b_ref, o_ref):
    x = x_ref[...]
    mu = jnp.mean(x, axis=-1, keepdims=True)
    var = jnp.mean((x - mu) ** 2, axis=-1, keepdims=True)
    o_ref[...] = (x - mu) * jax.lax.rsqrt(var + 1e-5) * g_ref[...] + b_ref[...]


def _layer_norm(x, g, b):
    n, d = x.shape
    rows = 1000
    return pl.pallas_call(
        _ln_kernel,
        grid=(n // rows,),
        in_specs=[
            pl.BlockSpec((rows, d), lambda i: (i, 0)),
            pl.BlockSpec((1, d), lambda i: (0, 0)),
            pl.BlockSpec((1, d), lambda i: (0, 0)),
        ],
        out_specs=pl.BlockSpec((rows, d), lambda i: (i, 0)),
        out_shape=jax.ShapeDtypeStruct((n, d), x.dtype),
    )(x, g.reshape(1, d), b.reshape(1, d))


def _gat_conv(x, src, dst, W, att_src, att_dst, bias, heads, out_ch, concat, n):
    h = (x @ W).reshape(n, heads, out_ch)
    a_src = jnp.sum(h * att_src, axis=-1)
    a_dst = jnp.sum(h * att_dst, axis=-1)
    alpha = a_src[src] + a_dst[dst]
    alpha = jax.nn.leaky_relu(alpha, negative_slope=0.2)
    amax = jax.ops.segment_max(alpha, dst, num_segments=n)
    amax = jnp.where(jnp.isfinite(amax), amax, 0.0)
    ex = jnp.exp(alpha - amax[dst])
    denom = jax.ops.segment_sum(ex, dst, num_segments=n)
    coef = ex / (denom[dst] + 1e-16)
    msg = h[src] * coef[..., None]
    out = jax.ops.segment_sum(msg, dst, num_segments=n)
    if concat:
        out = out.reshape(n, heads * out_ch)
    else:
        out = jnp.mean(out, axis=1)
    return out + bias


def _gcn_conv(x, src, dst, W, bias, dinv, n):
    h = x @ W
    norm = dinv[src] * dinv[dst]
    out = jax.ops.segment_sum(h[src] * norm[:, None], dst, num_segments=n)
    return out + bias


def kernel(x, edge_index, W_g1, att_src1, att_dst1, b_g1, W_g2, b_g2, W_g3, att_src3, att_dst3, b_g3, W_g4, b_g4, W_m1, b_m1, W_m2, b_m2, W_m3, b_m3, W_m4, b_m4, g0, be0, g1, be1, g2, be2):
    n = N_NODES
    ar = jnp.arange(n, dtype=edge_index.dtype)
    src = jnp.concatenate([edge_index[0], ar])
    dst = jnp.concatenate([edge_index[1], ar])

    ones = jnp.ones((src.shape[0],), dtype=x.dtype)
    deg = jax.ops.segment_sum(ones, dst, num_segments=n)
    dinv = jnp.where(deg > 0, jax.lax.rsqrt(deg), 0.0)

    h = _gat_conv(x, src, dst, W_g1, att_src1, att_dst1, b_g1, 8, 32, True, n) + (x @ W_m1 + b_m1)
    h = _layer_norm(h, g0, be0)
    h = _gcn_conv(h, src, dst, W_g2, b_g2, dinv, n) + (h @ W_m2 + b_m2)
    h = _layer_norm(h, g1, be1)
    h = _gat_conv(h, src, dst, W_g3, att_src3, att_dst3, b_g3, 8, 64, False, n) + (h @ W_m3 + b_m3)
    h = _layer_norm(h, g2, be2)
    out = _gcn_conv(h, src, dst, W_g4, b_g4, dinv, n) + (h @ W_m4 + b_m4)
    return out


# speedup vs baseline: 17.3421x; 17.3421x over previous
import functools

import jax
import jax.numpy as jnp
from jax import lax
from jax.experimental import pallas as pl
from jax.experimental.pallas import tpu as pltpu
from jax.experimental.pallas import tpu_sc as plsc

NN = 10000
NP = 10240
EE = 170000
EP = 172032
NC, NS, L = 2, 16, 16
CHUNK = 128
CPT = EP // (NS * CHUNK)
RPT = NP // NS

_mesh = plsc.VectorSubcoreMesh(core_axis_name="c", subcore_axis_name="s")
_f32 = jnp.float32


def _k1_body(x_ref, wg_ref, a_ref, wm_ref, bm_ref, h1_ref, asad_ref, mlp_ref):
    xb = x_ref[...]
    h = jnp.dot(xb, wg_ref[...], preferred_element_type=_f32)
    h1_ref[...] = h[None]
    contrib = jnp.dot(h, a_ref[0], preferred_element_type=_f32)
    jj = pl.program_id(1)

    @pl.when(jj == 0)
    def _():
        asad_ref[...] = contrib

    @pl.when(jj != 0)
    def _():
        asad_ref[...] = asad_ref[...] + contrib

    mlp_ref[...] = jnp.dot(xb, wm_ref[...], preferred_element_type=_f32) + bm_ref[...]


def _tc_layer1(x_pad, W_g1, A1, W_m1, b_m1):
    rows = 640
    grid = (NP // rows, 2)
    return pl.pallas_call(
        _k1_body,
        grid=grid,
        in_specs=[
            pl.BlockSpec((rows, 1024), lambda i, j: (i, 0)),
            pl.BlockSpec((1024, 128), lambda i, j: (0, j)),
            pl.BlockSpec((1, 128, 16), lambda i, j: (j, 0, 0)),
            pl.BlockSpec((1024, 128), lambda i, j: (0, j)),
            pl.BlockSpec((1, 128), lambda i, j: (0, j)),
        ],
        out_specs=[
            pl.BlockSpec((1, rows, 128), lambda i, j: (j, i, 0)),
            pl.BlockSpec((rows, 16), lambda i, j: (i, 0)),
            pl.BlockSpec((rows, 128), lambda i, j: (i, j)),
        ],
        out_shape=[
            jax.ShapeDtypeStruct((2, NP, 128), _f32),
            jax.ShapeDtypeStruct((NP, 16), _f32),
            jax.ShapeDtypeStruct((NP, 256), _f32),
        ],
    )(x_pad, W_g1, A1.reshape(2, 128, 16), W_m1, b_m1.reshape(1, 256))


def _ln_kernel(x_ref, g_ref, b_ref, o_ref):
    x = x_ref[...]
    mu = jnp.mean(x, axis=-1, keepdims=True)
    var = jnp.mean((x - mu) ** 2, axis=-1, keepdims=True)
    o_ref[...] = (x - mu) * jax.lax.rsqrt(var + 1e-5) * g_ref[...] + b_ref[...]


def _layer_norm(x, g, b):
    n, d = x.shape
    rows = 1000
    return pl.pallas_call(
        _ln_kernel,
        grid=(n // rows,),
        in_specs=[
            pl.BlockSpec((rows, d), lambda i: (i, 0)),
            pl.BlockSpec((1, d), lambda i: (0, 0)),
            pl.BlockSpec((1, d), lambda i: (0, 0)),
        ],
        out_specs=pl.BlockSpec((rows, d), lambda i: (i, 0)),
        out_shape=jax.ShapeDtypeStruct((n, d), x.dtype),
    )(x, g.reshape(1, d), b.reshape(1, d))


EPH_CPT = EP // (2 * NS * CHUNK)


def _sc_gcn_body(src_hbm, dst_hbm, tab_hbm, z_hbm, out_hbm,
                 sv, dv, hrows, acc, sem):
    s = lax.axis_index("c")
    t = lax.axis_index("s")
    pltpu.sync_copy(z_hbm.at[pl.ds(t * RPT, RPT)], acc.at[pl.ds(t * RPT, RPT)])
    plsc.subcore_barrier()

    def chunk(k, _):
        base = ((s * NS + t) * EPH_CPT + k) * CHUNK
        pltpu.sync_copy(src_hbm.at[pl.ds(base, CHUNK)], sv)
        pltpu.sync_copy(dst_hbm.at[pl.ds(base, CHUNK)], dv)
        pltpu.async_copy(tab_hbm.at[sv], hrows, sem).wait()
        pltpu.sync_copy(hrows, acc.at[dv], add=True)
        return ()

    lax.fori_loop(0, EPH_CPT, chunk, ())
    plsc.subcore_barrier()
    for u in range(RPT // CHUNK):
        r0 = t * RPT + u * CHUNK
        pltpu.sync_copy(acc.at[pl.ds(r0, CHUNK)], hrows)
        pltpu.sync_copy(hrows, out_hbm.at[pl.ds(s * NP + r0, CHUNK)])


def _sc_gcn(src_pad, dst_pad, tab, d):
    z = jnp.zeros((NP, d), _f32)
    kern = pl.kernel(
        _sc_gcn_body,
        out_type=jax.ShapeDtypeStruct((2 * NP, d), _f32),
        mesh=_mesh,
        scratch_types=[
            pltpu.VMEM((CHUNK,), jnp.int32),
            pltpu.VMEM((CHUNK,), jnp.int32),
            pltpu.VMEM((CHUNK, d), _f32),
            pltpu.VMEM_SHARED((NP, d), _f32),
            pltpu.SemaphoreType.DMA,
        ],
        compiler_params=pltpu.CompilerParams(use_tc_tiling_on_sc=False,
                                             needs_layout_passes=False),
    )
    return kern(src_pad, dst_pad, tab, z)


def _take16(vec, idx):
    return lax.gather(
        vec, idx[:, None],
        dimension_numbers=lax.GatherDimensionNumbers(
            offset_dims=(), collapsed_slice_dims=(0,), start_index_map=(0,)),
        slice_sizes=(1,),
        mode=lax.GatherScatterMode.PROMISE_IN_BOUNDS,
    )


def _sc_gat1_body(src_hbm, dst_hbm, asad_hbm, tab_hbm, z128_hbm, zden_hbm, zdeg_hbm,
                  out_hbm, deg_hbm,
                  sv, dv, gi, srow, drow, hrows, mb, exb, den2d, cfb, onesb, degbuf,
                  accs, dens, degs, sem):
    s = lax.axis_index("c")
    t = lax.axis_index("s")
    io = jnp.arange(L, dtype=jnp.int32)
    ioh = io // 8
    iom8 = io % 8
    cols_as = iom8 + (iom8 // 4) * 4
    cols_ad = cols_as + 4
    pim = "promise_in_bounds"

    pltpu.sync_copy(z128_hbm.at[pl.ds(t * RPT, RPT)], accs.at[pl.ds(t * RPT, RPT)])
    pltpu.sync_copy(zden_hbm.at[pl.ds(t * RPT, RPT)], dens.at[pl.ds(t * RPT, RPT)])
    pltpu.sync_copy(zdeg_hbm.at[pl.ds(t * RPT, RPT)], degs.at[pl.ds(t * RPT, RPT)])
    for j in range(CHUNK // L):
        onesb[pl.ds(j * L, L)] = jnp.full((L,), 1.0, _f32)
    plsc.subcore_barrier()

    def alpha_pair(j):
        rows = 2 * j + ioh
        sa = plsc.load_gather(srow, [rows, cols_as])
        da = plsc.load_gather(drow, [rows, cols_ad])
        al = sa + da
        al = jnp.where(al >= 0, al, al * jnp.float32(0.2))
        return jnp.exp(al)

    def phase_a(k, _):
        base = (t * CPT + k) * CHUNK
        pltpu.sync_copy(src_hbm.at[pl.ds(base, CHUNK)], sv)
        pltpu.sync_copy(dst_hbm.at[pl.ds(base, CHUNK)], dv)
        pltpu.async_copy(asad_hbm.at[sv], srow, sem).wait()
        pltpu.async_copy(asad_hbm.at[dv], drow, sem).wait()

        def pair(j, _):
            ex = alpha_pair(j)
            plsc.store_scatter(exb, [2 * j + ioh, iom8], ex)
            return ()
        lax.fori_loop(0, CHUNK // 2, pair, ())
        pltpu.sync_copy(exb, dens.at[dv], add=True)
        pltpu.sync_copy(onesb, degs.at[dv], add=True)
        return ()

    lax.fori_loop(0, CPT, phase_a, ())
    plsc.subcore_barrier()

    hb = 4 * s

    def phase_b(k, _):
        base = (t * CPT + k) * CHUNK
        pltpu.sync_copy(src_hbm.at[pl.ds(base, CHUNK)], sv)
        pltpu.sync_copy(dst_hbm.at[pl.ds(base, CHUNK)], dv)

        def gidx(j, _):
            gi[pl.ds(j * L, L)] = sv[pl.ds(j * L, L)] + s * NP
            return ()
        lax.fori_loop(0, CHUNK // L, gidx, (), unroll=True)
        pltpu.async_copy(asad_hbm.at[sv], srow, sem).wait()
        pltpu.async_copy(asad_hbm.at[dv], drow, sem).wait()
        pltpu.async_copy(dens.at[dv], den2d, sem).wait()
        pltpu.async_copy(tab_hbm.at[gi], hrows, sem).wait()

        def pair(j, _):
            ex = alpha_pair(j)
            den = plsc.load_gather(den2d, [2 * j + ioh, iom8])
            cfb[pl.ds(L * j, L)] = ex / (den + jnp.float32(1e-16))
            return ()
        lax.fori_loop(0, CHUNK // 2, pair, ())

        def msg_pair(p, _):
            c16 = cfb[pl.ds(L * p, L)]
            for q in range(2):
                e16 = jnp.full((L,), 2 * p + q, jnp.int32)
                for v in range(8):
                    cv = _take16(c16, jnp.full((L,), 8 * q + v // 2, jnp.int32) + hb)
                    off = 16 * v + io
                    hv = plsc.load_gather(hrows, [e16, off])
                    plsc.store_scatter(mb, [e16, off], hv * cv)
            return ()
        lax.fori_loop(0, CHUNK // 2, msg_pair, ())
        pltpu.sync_copy(mb, accs.at[dv], add=True)
        return ()

    lax.fori_loop(0, CPT, phase_b, ())
    plsc.subcore_barrier()

    for u in range(RPT // CHUNK):
        r0 = t * RPT + u * CHUNK
        pltpu.sync_copy(accs.at[pl.ds(r0, CHUNK)], mb)
        pltpu.sync_copy(mb, out_hbm.at[pl.ds(s * NP + r0, CHUNK)])

    @pl.when(s == 0)
    def _():
        pltpu.sync_copy(degs.at[pl.ds(t * RPT, RPT)], degbuf)
        pltpu.sync_copy(degbuf, deg_hbm.at[pl.ds(t * RPT, RPT)])


def _sc_gat1(src_pad, dst_pad, asad, tab):
    z128 = jnp.zeros((NP, 128), _f32)
    zden = jnp.zeros((NP, 8), _f32)
    zdeg = jnp.zeros((NP,), _f32)
    kern = pl.kernel(
        _sc_gat1_body,
        out_type=[
            jax.ShapeDtypeStruct((2 * NP, 128), _f32),
            jax.ShapeDtypeStruct((NP,), _f32),
        ],
        mesh=_mesh,
        scratch_types=[
            pltpu.VMEM((CHUNK,), jnp.int32),
            pltpu.VMEM((CHUNK,), jnp.int32),
            pltpu.VMEM((CHUNK,), jnp.int32),
            pltpu.VMEM((CHUNK, 16), _f32),
            pltpu.VMEM((CHUNK, 16), _f32),
            pltpu.VMEM((CHUNK, 128), _f32),
            pltpu.VMEM((CHUNK, 128), _f32),
            pltpu.VMEM((CHUNK, 8), _f32),
            pltpu.VMEM((CHUNK, 8), _f32),
            pltpu.VMEM((CHUNK * 8,), _f32),
            pltpu.VMEM((CHUNK,), _f32),
            pltpu.VMEM((RPT,), _f32),
            pltpu.VMEM_SHARED((NP, 128), _f32),
            pltpu.VMEM_SHARED((NP, 8), _f32),
            pltpu.VMEM_SHARED((NP,), _f32),
            pltpu.SemaphoreType.DMA,
        ],
        compiler_params=pltpu.CompilerParams(use_tc_tiling_on_sc=False,
                                             needs_layout_passes=False),
    )
    return kern(src_pad, dst_pad, asad, tab, z128, zden, zdeg)


CH3 = 64
CPT3 = EP // (2 * NS * CH3)


def _sc_gat3_body(src_hbm, dst_hbm, asad_hbm, tab_hbm, z64_hbm, zden_hbm,
                  out_hbm,
                  sv, dv, svb, dvb, srow, drow, srb, drb, hrows, mb, exb,
                  den2d, cfb, accs, dens, sem):
    s = lax.axis_index("c")
    t = lax.axis_index("s")
    io = jnp.arange(L, dtype=jnp.int32)
    ioh = io // 8
    iom8 = io % 8
    cols_as = iom8 + (iom8 // 4) * 4
    cols_ad = cols_as + 4

    pltpu.sync_copy(z64_hbm.at[pl.ds(t * RPT, RPT)], accs.at[pl.ds(t * RPT, RPT)])
    pltpu.sync_copy(zden_hbm.at[pl.ds(t * RPT, RPT)], dens.at[pl.ds(t * RPT, RPT)])
    plsc.subcore_barrier()

    def alpha_pair(sr, dr, j):
        rows = 2 * j + ioh
        sa = plsc.load_gather(sr, [rows, cols_as])
        da = plsc.load_gather(dr, [rows, cols_ad])
        al = sa + da
        al = jnp.where(al >= 0, al, al * jnp.float32(0.2))
        return jnp.exp(al)

    def phase_a(k, _):
        base = (t * CPT + k) * CHUNK
        pltpu.sync_copy(src_hbm.at[pl.ds(base, CHUNK)], sv)
        pltpu.sync_copy(dst_hbm.at[pl.ds(base, CHUNK)], dv)
        pltpu.async_copy(asad_hbm.at[sv], srow, sem).wait()
        pltpu.async_copy(asad_hbm.at[dv], drow, sem).wait()

        def pair(j, _):
            ex = alpha_pair(srow, drow, j)
            plsc.store_scatter(exb, [2 * j + ioh, iom8], ex)
            return ()
        lax.fori_loop(0, CHUNK // 2, pair, ())
        pltpu.sync_copy(exb, dens.at[dv], add=True)
        return ()

    lax.fori_loop(0, CPT, phase_a, ())
    plsc.subcore_barrier()

    def phase_b(k, _):
        base = ((s * NS + t) * CPT3 + k) * CH3
        pltpu.sync_copy(src_hbm.at[pl.ds(base, CH3)], svb)
        pltpu.sync_copy(dst_hbm.at[pl.ds(base, CH3)], dvb)
        pltpu.async_copy(asad_hbm.at[svb], srb, sem).wait()
        pltpu.async_copy(asad_hbm.at[dvb], drb, sem).wait()
        pltpu.async_copy(dens.at[dvb], den2d, sem).wait()
        pltpu.async_copy(tab_hbm.at[svb], hrows, sem).wait()

        def pair(j, _):
            ex = alpha_pair(srb, drb, j)
            den = plsc.load_gather(den2d, [2 * j + ioh, iom8])
            cfb[pl.ds(L * j, L)] = ex / (den + jnp.float32(1e-16)) * jnp.float32(0.125)
            return ()
        lax.fori_loop(0, CH3 // 2, pair, ())

        def msg_pair(p, _):
            c16 = cfb[pl.ds(L * p, L)]
            for q in range(2):
                e16 = jnp.full((L,), 2 * p + q, jnp.int32)
                cvs = [_take16(c16, jnp.full((L,), 8 * q + h, jnp.int32))
                       for h in range(8)]
                for v in range(4):
                    acc = cvs[0] * plsc.load_gather(hrows, [e16, 16 * v + io])
                    for h in range(1, 8):
                        acc = acc + cvs[h] * plsc.load_gather(
                            hrows, [e16, 64 * h + 16 * v + io])
                    plsc.store_scatter(mb, [e16, 16 * v + io], acc)
            return ()
        lax.fori_loop(0, CH3 // 2, msg_pair, ())
        pltpu.sync_copy(mb, accs.at[dvb], add=True)
        return ()

    lax.fori_loop(0, CPT3, phase_b, ())
    plsc.subcore_barrier()
    for u in range(RPT // CH3):
        r0 = t * RPT + u * CH3
        pltpu.sync_copy(accs.at[pl.ds(r0, CH3)], mb)
        pltpu.sync_copy(mb, out_hbm.at[pl.ds(s * NP + r0, CH3)])


def _sc_gat3(src_pad, dst_pad, asad, tab):
    z64 = jnp.zeros((NP, 64), _f32)
    zden = jnp.zeros((NP, 8), _f32)
    kern = pl.kernel(
        _sc_gat3_body,
        out_type=jax.ShapeDtypeStruct((2 * NP, 64), _f32),
        mesh=_mesh,
        scratch_types=[
            pltpu.VMEM((CHUNK,), jnp.int32),
            pltpu.VMEM((CHUNK,), jnp.int32),
            pltpu.VMEM((CH3,), jnp.int32),
            pltpu.VMEM((CH3,), jnp.int32),
            pltpu.VMEM((CHUNK, 16), _f32),
            pltpu.VMEM((CHUNK, 16), _f32),
            pltpu.VMEM((CH3, 16), _f32),
            pltpu.VMEM((CH3, 16), _f32),
            pltpu.VMEM((CH3, 512), _f32),
            pltpu.VMEM((CH3, 64), _f32),
            pltpu.VMEM((CHUNK, 8), _f32),
            pltpu.VMEM((CH3, 8), _f32),
            pltpu.VMEM((CH3 * 8,), _f32),
            pltpu.VMEM_SHARED((NP, 64), _f32),
            pltpu.VMEM_SHARED((NP, 8), _f32),
            pltpu.SemaphoreType.DMA,
        ],
        compiler_params=pltpu.CompilerParams(use_tc_tiling_on_sc=False,
                                             needs_layout_passes=False),
    )
    return kern(src_pad, dst_pad, asad, tab, z64, zden)


def _gat_conv_jnp(h_flat, as1, ad1, src, dst, bias, heads, out_ch, concat, n):
    h = h_flat.reshape(n, heads, out_ch)
    alpha = as1[src] + ad1[dst]
    alpha = jax.nn.leaky_relu(alpha, negative_slope=0.2)
    amax = jax.ops.segment_max(alpha, dst, num_segments=n)
    amax = jnp.where(jnp.isfinite(amax), amax, 0.0)
    ex = jnp.exp(alpha - amax[dst])
    denom = jax.ops.segment_sum(ex, dst, num_segments=n)
    coef = ex / (denom[dst] + 1e-16)
    msg = h[src] * coef[..., None]
    out = jax.ops.segment_sum(msg, dst, num_segments=n)
    if concat:
        out = out.reshape(n, heads * out_ch)
    else:
        out = jnp.mean(out, axis=1)
    return out + bias


def _build_A(att_src, att_dst, heads, out_ch):
    d = heads * out_ch
    A = jnp.zeros((d, 16), _f32)
    asf = att_src.reshape(-1)
    adf = att_dst.reshape(-1)
    for h in range(heads):
        half = h // 4
        sl = slice(h * out_ch, (h + 1) * out_ch)
        A = A.at[sl, 8 * half + (h % 4)].set(asf[sl])
        A = A.at[sl, 8 * half + 4 + (h % 4)].set(adf[sl])
    return A


def kernel(x, edge_index, W_g1, att_src1, att_dst1, b_g1, W_g2, b_g2, W_g3, att_src3, att_dst3, b_g3, W_g4, b_g4, W_m1, b_m1, W_m2, b_m2, W_m3, b_m3, W_m4, b_m4, g0, be0, g1, be1, g2, be2):
    n = NN
    ei = edge_index.astype(jnp.int32)
    ar = jnp.arange(n, dtype=jnp.int32)
    pad = NN + (jnp.arange(EP - EE, dtype=jnp.int32) % (NP - NN))
    src_pad = jnp.concatenate([ei[0], ar, pad])
    dst_pad = jnp.concatenate([ei[1], ar, pad])
    src = src_pad[:EE]
    dst = dst_pad[:EE]

    x_pad = jnp.pad(x, ((0, NP - NN), (0, 0)))
    A1 = _build_A(att_src1, att_dst1, 8, 32)
    h1r, asad1, mlp1p = _tc_layer1(x_pad, W_g1, A1, W_m1, b_m1)

    mlp1 = mlp1p[:NN]

    gat1r, degp = _sc_gat1(src_pad, dst_pad, asad1, h1r.reshape(2 * NP, 128))
    deg = degp[:NN]
    dinv = jnp.where(deg > 0, jax.lax.rsqrt(deg), 0.0)
    gat1 = jnp.concatenate([gat1r[:NP][:NN], gat1r[NP:][:NN]], axis=1)

    h = gat1 + b_g1 + mlp1
    h = _layer_norm(h, g0, be0)

    h2m = h @ W_g2
    hs2p = jnp.pad(dinv[:, None] * h2m, ((0, NP - NN), (0, 0)))
    gcn2r = _sc_gcn(src_pad, dst_pad, hs2p, 128)
    gcn2 = dinv[:, None] * (gcn2r[:NP] + gcn2r[NP:])[:NN] + b_g2
    h = gcn2 + (h @ W_m2 + b_m2)
    h = _layer_norm(h, g1, be1)

    h3 = h @ W_g3
    as3 = jnp.sum(h3.reshape(n, 8, 64) * att_src3, axis=-1)
    ad3 = jnp.sum(h3.reshape(n, 8, 64) * att_dst3, axis=-1)
    asad3 = jnp.concatenate([as3[:, :4], ad3[:, :4], as3[:, 4:], ad3[:, 4:]], axis=1)
    asad3 = jnp.pad(asad3, ((0, NP - NN), (0, 0)))
    tab3 = jnp.pad(h3, ((0, NP - NN), (0, 0)))
    out3r = _sc_gat3(src_pad, dst_pad, asad3, tab3)
    gat3 = (out3r[:NP] + out3r[NP:])[:NN] + b_g3
    h = gat3 + (h @ W_m3 + b_m3)
    h = _layer_norm(h, g2, be2)

    tab4 = jnp.pad(dinv[:, None] * h, ((0, NP - NN), (0, 0)))
    s4r = _sc_gcn(src_pad, dst_pad, tab4, 64)
    s4 = (s4r[:NP] + s4r[NP:])[:NN]
    out = dinv[:, None] * (s4 @ W_g4) + b_g4 + (h @ W_m4 + b_m4)
    return out
